# baseline (speedup 1.0000x reference)
import functools

import jax
import jax.numpy as jnp
from jax import lax
from jax.experimental import pallas as pl
from jax.experimental.pallas import tpu as pltpu
from jax.experimental.pallas import tpu_sc as plsc

_NC = 2
_NS = 16
_NW = _NC * _NS
_L = 16
_TOK = 128
_EUNROLL = 8


def _position_encoding(seq_len, hidden, start, dtype):
    power = jnp.arange(0, hidden, 2, dtype=dtype) / hidden
    divisor = 10000.0 ** power
    seqpos = jnp.arange(start, seq_len + start, dtype=dtype)
    index = seqpos[:, None] / divisor[None, :]
    pe = jnp.stack((jnp.sin(index), jnp.cos(index)), axis=-1)
    return pe.reshape(seq_len, hidden)


def _body(idx_hbm, par_hbm, table_hbm, pe_hbm, out_hbm,
          idx_v, par_v, pe_v, g_v, o_v, gsem, osem):
    seq, nchunk, tok = idx_hbm.shape
    embed = pe_hbm.shape[1] // _L
    ntask = seq * nchunk
    tw = ntask // _NW
    rows = 16
    nb16 = tok // _L

    wid = lax.axis_index("s") * _NC + lax.axis_index("c")
    t0 = wid * tw
    row_base = jnp.minimum((t0 // nchunk) // 8 * 8, seq - rows)

    pltpu.sync_copy(idx_hbm.at[pl.ds(row_base, rows)], idx_v)
    pltpu.sync_copy(par_hbm.at[pl.ds(row_base, rows)], par_v)
    pltpu.sync_copy(pe_hbm.at[pl.ds(row_base, rows)], pe_v)

    ridx = [lax.iota(jnp.int32, _L) + (b * _L) for b in range(nb16)]

    def issue_gather(t, buf):
        s = t // nchunk
        c = lax.rem(t, nchunk)
        pltpu.async_copy(
            table_hbm.at[idx_v.at[s - row_base, c]], g_v.at[buf], gsem.at[buf]
        )

    def process(t, buf, first):
        s = t // nchunk
        c = lax.rem(t, nchunk)
        sr = s - row_base
        pltpu.make_async_copy(
            table_hbm.at[idx_v.at[sr, c]], g_v.at[buf], gsem.at[buf]
        ).wait()

        @pl.when(jnp.logical_not(first))
        def _drain_out():
            t2 = t - 2
            s2 = t2 // nchunk
            c2 = lax.rem(t2, nchunk)
            pltpu.make_async_copy(
                o_v.at[buf],
                out_hbm.at[s2, :, pl.ds(c2 * tok, tok)],
                osem.at[buf],
            ).wait()

        pcol = [par_v[sr, c, pl.ds(b * _L, _L)] for b in range(nb16)]

        def e_body(e4, carry):
            for de in range(_EUNROLL):
                e = e4 * _EUNROLL + de
                pv = pe_v[sr, pl.ds(e * _L, _L)]
                for b in range(nb16):
                    vals = plsc.load_gather(g_v.at[buf], [ridx[b], pcol[b] + e])
                    o_v[buf, e, pl.ds(b * _L, _L)] = vals + pv
            return carry

        lax.fori_loop(0, embed // _EUNROLL, e_body, 0)
        pltpu.async_copy(
            o_v.at[buf], out_hbm.at[s, :, pl.ds(c * tok, tok)], osem.at[buf]
        )

    issue_gather(t0, 0)

    def loop_body(k, carry):
        t = t0 + 2 * k
        issue_gather(t + 1, 1)
        process(t, 0, k == 0)

        @pl.when(k < tw // 2 - 1)
        def _next_g():
            issue_gather(t + 2, 0)

        process(t + 1, 1, k == 0)
        return carry

    lax.fori_loop(0, tw // 2, loop_body, 0)

    def drain_body(j, carry):
        t = t0 + j
        buf = lax.rem(t, 2)
        s = t // nchunk
        c = lax.rem(t, nchunk)
        pltpu.make_async_copy(
            o_v.at[buf], out_hbm.at[s, :, pl.ds(c * tok, tok)], osem.at[buf]
        ).wait()
        return carry

    lax.fori_loop(tw - 2, tw, drain_body, 0)


def kernel(inputs, embedding_table):
    batch, seq = inputs.shape
    nsym, embed = embedding_table.shape
    nchunk = batch // _TOK
    pe = _position_encoding(seq, embed, 1, embedding_table.dtype)
    pe_rep = jnp.broadcast_to(pe[:, :, None], (seq, embed, _L)).reshape(
        seq, embed * _L
    )
    idx_t = inputs.T.reshape(seq, nchunk, _TOK)
    idx_half = idx_t >> 1
    par64 = (idx_t & 1) * embed
    table2 = embedding_table.reshape(nsym // 2, embed * 2)
    rows = 16

    mesh = plsc.VectorSubcoreMesh(
        core_axis_name="c", subcore_axis_name="s", num_cores=_NC, num_subcores=_NS
    )
    run = pl.kernel(
        _body,
        out_type=jax.ShapeDtypeStruct((seq, embed, batch), embedding_table.dtype),
        mesh=mesh,
        scratch_types=[
            pltpu.VMEM((rows, nchunk, _TOK), jnp.int32),
            pltpu.VMEM((rows, nchunk, _TOK), jnp.int32),
            pltpu.VMEM((rows, embed * _L), jnp.float32),
            pltpu.VMEM((2, _TOK, embed * 2), jnp.float32),
            pltpu.VMEM((2, embed, _TOK), jnp.float32),
            pltpu.SemaphoreType.DMA((2,)),
            pltpu.SemaphoreType.DMA((2,)),
        ],
        compiler_params=pltpu.CompilerParams(
            use_tc_tiling_on_sc=True, needs_layout_passes=False,
            disable_bounds_checks=True
        ),
    )
    out = run(idx_half, par64, table2, pe_rep)
    return jnp.transpose(out, (2, 0, 1))


# speedup vs baseline: 1.1181x; 1.1181x over previous
import functools

import jax
import jax.numpy as jnp
from jax import lax
from jax.experimental import pallas as pl
from jax.experimental.pallas import tpu as pltpu
from jax.experimental.pallas import tpu_sc as plsc

_NC = 2
_NS = 16
_NW = _NC * _NS
_CHUNK = 100
_NBUF = 4


def _position_encoding(seq_len, hidden, start, dtype):
    power = jnp.arange(0, hidden, 2, dtype=dtype) / hidden
    divisor = 10000.0 ** power
    seqpos = jnp.arange(start, seq_len + start, dtype=dtype)
    index = seqpos[:, None] / divisor[None, :]
    pe = jnp.stack((jnp.sin(index), jnp.cos(index)), axis=-1)
    return pe.reshape(seq_len, hidden)


def _body(idx_hbm, table_hbm, pe_hbm, out_hbm, idx_v, rows_v, isem, gsem, osem):
    nb = idx_hbm.shape[0] // _NW
    seq = pe_hbm.shape[0]
    nchunk = seq // _CHUNK
    wid = lax.axis_index("s") * _NC + lax.axis_index("c")
    base = wid * nb

    pltpu.sync_copy(idx_hbm.at[pl.ds(base, nb)], idx_v)

    def stage_a(i):
        buf = lax.rem(i, _NBUF)

        @pl.when(i >= _NBUF)
        def _drain_scatter():
            pltpu.make_async_copy(
                rows_v.at[buf], out_hbm.at[base + i - _NBUF], osem.at[buf]
            ).wait()

        pltpu.async_copy(pe_hbm, rows_v.at[buf], isem.at[buf])

    def stage_b(i):
        buf = lax.rem(i, _NBUF)
        pltpu.make_async_copy(pe_hbm, rows_v.at[buf], isem.at[buf]).wait()
        for j in range(nchunk):
            pltpu.async_copy(
                table_hbm.at[idx_v.at[i, j]],
                rows_v.at[buf, pl.ds(j * _CHUNK, _CHUNK)],
                gsem.at[buf],
                add=True,
            )

    def stage_c(i):
        buf = lax.rem(i, _NBUF)
        for j in range(nchunk):
            pltpu.make_async_copy(
                table_hbm.at[idx_v.at[i, j]],
                rows_v.at[buf, pl.ds(j * _CHUNK, _CHUNK)],
                gsem.at[buf],
            ).wait()
        pltpu.async_copy(rows_v.at[buf], out_hbm.at[base + i], osem.at[buf])

    def loop_body(i, carry):
        @pl.when(i < nb)
        def _a():
            stage_a(i)

        @pl.when(jnp.logical_and(i >= 1, i <= nb))
        def _b():
            stage_b(i - 1)

        @pl.when(i >= 2)
        def _c():
            stage_c(i - 2)

        return carry

    lax.fori_loop(0, nb + 2, loop_body, 0)

    def drain_body(i, carry):
        buf = lax.rem(i, _NBUF)
        pltpu.make_async_copy(
            rows_v.at[buf], out_hbm.at[base + i], osem.at[buf]
        ).wait()
        return carry

    lax.fori_loop(nb - _NBUF, nb, drain_body, 0)


def kernel(inputs, embedding_table):
    batch, seq = inputs.shape
    _, embed = embedding_table.shape
    pe = _position_encoding(seq, embed, 1, embedding_table.dtype)
    idx = inputs.reshape(batch, seq // _CHUNK, _CHUNK)

    mesh = plsc.VectorSubcoreMesh(
        core_axis_name="c", subcore_axis_name="s", num_cores=_NC, num_subcores=_NS
    )
    run = pl.kernel(
        _body,
        out_type=jax.ShapeDtypeStruct((batch, seq, embed), embedding_table.dtype),
        mesh=mesh,
        scratch_types=[
            pltpu.VMEM((batch // _NW, seq // _CHUNK, _CHUNK), jnp.int32),
            pltpu.VMEM((_NBUF, seq, embed), jnp.float32),
            pltpu.SemaphoreType.DMA((_NBUF,)),
            pltpu.SemaphoreType.DMA((_NBUF,)),
            pltpu.SemaphoreType.DMA((_NBUF,)),
        ],
        compiler_params=pltpu.CompilerParams(use_tc_tiling_on_sc=False),
    )
    return run(idx, embedding_table, pe)


# speedup vs baseline: 1.1200x; 1.0018x over previous
import functools

import jax
import jax.numpy as jnp
from jax import lax
from jax.experimental import pallas as pl
from jax.experimental.pallas import tpu as pltpu
from jax.experimental.pallas import tpu_sc as plsc

_NC = 2
_NS = 16
_NW = _NC * _NS
_CHUNK = 100
_NBUF = 6


def _position_encoding(seq_len, hidden, start, dtype):
    power = jnp.arange(0, hidden, 2, dtype=dtype) / hidden
    divisor = 10000.0 ** power
    seqpos = jnp.arange(start, seq_len + start, dtype=dtype)
    index = seqpos[:, None] / divisor[None, :]
    pe = jnp.stack((jnp.sin(index), jnp.cos(index)), axis=-1)
    return pe.reshape(seq_len, hidden)


def _body(idx_hbm, table_hbm, pe_hbm, out_hbm, idx_v, rows_v, isem, gsem, osem):
    nb = idx_hbm.shape[0] // _NW
    seq = pe_hbm.shape[0]
    nchunk = seq // _CHUNK
    wid = lax.axis_index("s") * _NC + lax.axis_index("c")
    base = wid * nb

    pltpu.sync_copy(idx_hbm.at[pl.ds(base, nb)], idx_v)

    def stage_a(i):
        buf = lax.rem(i, _NBUF)

        @pl.when(i >= _NBUF)
        def _drain_scatter():
            pltpu.make_async_copy(
                rows_v.at[buf], out_hbm.at[base + i - _NBUF], osem.at[buf]
            ).wait()

        pltpu.async_copy(pe_hbm, rows_v.at[buf], isem.at[buf])

    def stage_b(i):
        buf = lax.rem(i, _NBUF)
        pltpu.make_async_copy(pe_hbm, rows_v.at[buf], isem.at[buf]).wait()
        for j in range(nchunk):
            pltpu.async_copy(
                table_hbm.at[idx_v.at[i, j]],
                rows_v.at[buf, pl.ds(j * _CHUNK, _CHUNK)],
                gsem.at[buf],
                add=True,
            )

    def stage_c(i):
        buf = lax.rem(i, _NBUF)
        for j in range(nchunk):
            pltpu.make_async_copy(
                table_hbm.at[idx_v.at[i, j]],
                rows_v.at[buf, pl.ds(j * _CHUNK, _CHUNK)],
                gsem.at[buf],
            ).wait()
        pltpu.async_copy(rows_v.at[buf], out_hbm.at[base + i], osem.at[buf])

    def loop_body(i, carry):
        @pl.when(i < nb)
        def _a():
            stage_a(i)

        @pl.when(jnp.logical_and(i >= 1, i <= nb))
        def _b():
            stage_b(i - 1)

        @pl.when(i >= 2)
        def _c():
            stage_c(i - 2)

        return carry

    lax.fori_loop(0, nb + 2, loop_body, 0)

    def drain_body(i, carry):
        buf = lax.rem(i, _NBUF)
        pltpu.make_async_copy(
            rows_v.at[buf], out_hbm.at[base + i], osem.at[buf]
        ).wait()
        return carry

    lax.fori_loop(nb - _NBUF, nb, drain_body, 0)


def kernel(inputs, embedding_table):
    batch, seq = inputs.shape
    _, embed = embedding_table.shape
    pe = _position_encoding(seq, embed, 1, embedding_table.dtype)
    idx = inputs.reshape(batch, seq // _CHUNK, _CHUNK)

    mesh = plsc.VectorSubcoreMesh(
        core_axis_name="c", subcore_axis_name="s", num_cores=_NC, num_subcores=_NS
    )
    run = pl.kernel(
        _body,
        out_type=jax.ShapeDtypeStruct((batch, seq, embed), embedding_table.dtype),
        mesh=mesh,
        scratch_types=[
            pltpu.VMEM((batch // _NW, seq // _CHUNK, _CHUNK), jnp.int32),
            pltpu.VMEM((_NBUF, seq, embed), jnp.float32),
            pltpu.SemaphoreType.DMA((_NBUF,)),
            pltpu.SemaphoreType.DMA((_NBUF,)),
            pltpu.SemaphoreType.DMA((_NBUF,)),
        ],
        compiler_params=pltpu.CompilerParams(use_tc_tiling_on_sc=False),
    )
    return run(idx, embedding_table, pe)
